# Initial kernel scaffold; baseline (speedup 1.0000x reference)
import jax
import jax.numpy as jnp
from jax.experimental import pallas as pl


def kernel(x, W, b):
    raise NotImplementedError("write your pallas kernel here")


# speedup vs baseline: 2.0407x; 2.0407x over previous
import functools

import jax
import jax.numpy as jnp
from jax.experimental import pallas as pl

K = 20
DP = 128
BR = 256
NEG = -3.0e38


def _attr_kernel(x_ref, w_ref, b_ref, out_ref):
    z = jnp.dot(x_ref[...], w_ref[...], preferred_element_type=jnp.float32)
    out_ref[...] = jax.nn.sigmoid(z + b_ref[...])


def _maxval_kernel(ab_ref, aall_ref, out_ref):
    a = jax.lax.dot_general(
        ab_ref[...], aall_ref[...],
        (((1,), (1,)), ((), ())),
        preferred_element_type=jnp.float32,
    )
    pmax = jnp.max(a, axis=0, keepdims=True)

    @pl.when(pl.program_id(0) == 0)
    def _():
        out_ref[...] = pmax

    @pl.when(pl.program_id(0) != 0)
    def _():
        out_ref[...] = jnp.maximum(out_ref[...], pmax)


def _topk_kernel(ab_ref, aall_ref, mv_ref, vals_ref, idx_ref, *, n, br):
    a = jax.lax.dot_general(
        ab_ref[...], aall_ref[...],
        (((1,), (1,)), ((), ())),
        preferred_element_type=jnp.float32,
    )
    slab = a / mv_ref[...]
    iota = jax.lax.broadcasted_iota(jnp.int32, (br, n), 1)
    vals = []
    idxs = []
    for _ in range(K):
        m = jnp.max(slab, axis=1, keepdims=True)
        cand = jnp.where(slab == m, iota, n)
        ix = jnp.min(cand, axis=1, keepdims=True)
        vals.append(m)
        idxs.append(ix)
        slab = jnp.where(cand == ix, NEG, slab)
    vals_ref[...] = jnp.concatenate(vals, axis=1)
    idx_ref[...] = jnp.concatenate(idxs, axis=1)


def kernel(x, W, b):
    n, d_in = x.shape
    d_attr = W.shape[1]
    br = BR if n % BR == 0 else n
    nblk = n // br

    w_pad = jnp.zeros((d_in, DP), jnp.float32).at[:, :d_attr].set(W)
    b_pad = jnp.full((1, DP), -1e30, jnp.float32).at[0, :d_attr].set(b)

    attr = pl.pallas_call(
        _attr_kernel,
        grid=(nblk,),
        in_specs=[
            pl.BlockSpec((br, d_in), lambda i: (i, 0)),
            pl.BlockSpec((d_in, DP), lambda i: (0, 0)),
            pl.BlockSpec((1, DP), lambda i: (0, 0)),
        ],
        out_specs=pl.BlockSpec((br, DP), lambda i: (i, 0)),
        out_shape=jax.ShapeDtypeStruct((n, DP), jnp.float32),
    )(x, w_pad, b_pad)

    maxval = pl.pallas_call(
        _maxval_kernel,
        grid=(nblk,),
        in_specs=[
            pl.BlockSpec((br, DP), lambda i: (i, 0)),
            pl.BlockSpec((n, DP), lambda i: (0, 0)),
        ],
        out_specs=pl.BlockSpec((1, n), lambda i: (0, 0)),
        out_shape=jax.ShapeDtypeStruct((1, n), jnp.float32),
    )(attr, attr)

    vals, idxs = pl.pallas_call(
        functools.partial(_topk_kernel, n=n, br=br),
        grid=(nblk,),
        in_specs=[
            pl.BlockSpec((br, DP), lambda i: (i, 0)),
            pl.BlockSpec((n, DP), lambda i: (0, 0)),
            pl.BlockSpec((1, n), lambda i: (0, 0)),
        ],
        out_specs=[
            pl.BlockSpec((br, K), lambda i: (i, 0)),
            pl.BlockSpec((br, K), lambda i: (i, 0)),
        ],
        out_shape=[
            jax.ShapeDtypeStruct((n, K), jnp.float32),
            jax.ShapeDtypeStruct((n, K), jnp.int32),
        ],
    )(attr, attr, maxval)

    values = vals.reshape(-1)
    rows = jnp.repeat(jnp.arange(n, dtype=jnp.int32), K)
    edge_index = jnp.stack([rows, idxs.reshape(-1)], axis=0)
    return values, edge_index


# speedup vs baseline: 2.0720x; 1.0154x over previous
import functools

import jax
import jax.numpy as jnp
from jax.experimental import pallas as pl
from jax.experimental.pallas import tpu as pltpu

K = 20
DP = 128
BR = 256
NEG = -3.0e38


def _attr_kernel(x_ref, w_ref, b_ref, out_ref):
    z = jnp.dot(x_ref[...], w_ref[...], preferred_element_type=jnp.float32)
    out_ref[...] = jax.nn.sigmoid(z + b_ref[...])


def _maxval_kernel(ab_ref, aall_ref, out_ref):
    a = jax.lax.dot_general(
        ab_ref[...], aall_ref[...],
        (((1,), (1,)), ((), ())),
        preferred_element_type=jnp.float32,
    )
    pmax = jnp.max(a, axis=0, keepdims=True)

    @pl.when(pl.program_id(0) == 0)
    def _():
        out_ref[...] = pmax

    @pl.when(pl.program_id(0) != 0)
    def _():
        out_ref[...] = jnp.maximum(out_ref[...], pmax)


def _topk_kernel(aall_ref, mv_ref, vals_ref, idx_ref, slab_ref, *, n, br, nblk):
    i = pl.program_id(0)

    @pl.when(i < nblk)
    def _():
        ab = aall_ref[pl.ds(i * br, br), :]
        a = jax.lax.dot_general(
            ab, aall_ref[...],
            (((1,), (1,)), ((), ())),
            preferred_element_type=jnp.float32,
        )
        off = (i % 2) * br
        slab_ref[pl.ds(off, br), :] = a / mv_ref[...]

    @pl.when(i > 0)
    def _():
        off = ((i - 1) % 2) * br
        iota = jax.lax.broadcasted_iota(jnp.int32, (br, n), 1)
        kiota = jax.lax.broadcasted_iota(jnp.int32, (br, K), 1)

        def body(k, acc):
            vacc, iacc = acc
            slab = slab_ref[pl.ds(off, br), :]
            m = jnp.max(slab, axis=1, keepdims=True)
            cand = jnp.where(slab == m, iota, n)
            ix = jnp.min(cand, axis=1, keepdims=True)
            slab_ref[pl.ds(off, br), :] = jnp.where(cand == ix, NEG, slab)
            vacc = jnp.where(kiota == k, m, vacc)
            iacc = jnp.where(kiota == k, ix, iacc)
            return vacc, iacc

        vacc, iacc = jax.lax.fori_loop(
            0, K, body,
            (jnp.zeros((br, K), jnp.float32), jnp.zeros((br, K), jnp.int32)))
        vals_ref[...] = vacc
        idx_ref[...] = iacc


def kernel(x, W, b):
    n, d_in = x.shape
    d_attr = W.shape[1]
    br = BR if n % BR == 0 else n
    nblk = n // br

    w_pad = jnp.zeros((d_in, DP), jnp.float32).at[:, :d_attr].set(W)
    b_pad = jnp.full((1, DP), -1e30, jnp.float32).at[0, :d_attr].set(b)

    attr = pl.pallas_call(
        _attr_kernel,
        grid=(nblk,),
        in_specs=[
            pl.BlockSpec((br, d_in), lambda i: (i, 0)),
            pl.BlockSpec((d_in, DP), lambda i: (0, 0)),
            pl.BlockSpec((1, DP), lambda i: (0, 0)),
        ],
        out_specs=pl.BlockSpec((br, DP), lambda i: (i, 0)),
        out_shape=jax.ShapeDtypeStruct((n, DP), jnp.float32),
    )(x, w_pad, b_pad)

    maxval = pl.pallas_call(
        _maxval_kernel,
        grid=(nblk,),
        in_specs=[
            pl.BlockSpec((br, DP), lambda i: (i, 0)),
            pl.BlockSpec((n, DP), lambda i: (0, 0)),
        ],
        out_specs=pl.BlockSpec((1, n), lambda i: (0, 0)),
        out_shape=jax.ShapeDtypeStruct((1, n), jnp.float32),
    )(attr, attr)

    vals, idxs = pl.pallas_call(
        functools.partial(_topk_kernel, n=n, br=br, nblk=nblk),
        grid=(nblk + 1,),
        in_specs=[
            pl.BlockSpec((n, DP), lambda i: (0, 0)),
            pl.BlockSpec((1, n), lambda i: (0, 0)),
        ],
        out_specs=[
            pl.BlockSpec((br, K), lambda i: (jnp.maximum(i - 1, 0), 0)),
            pl.BlockSpec((br, K), lambda i: (jnp.maximum(i - 1, 0), 0)),
        ],
        out_shape=[
            jax.ShapeDtypeStruct((n, K), jnp.float32),
            jax.ShapeDtypeStruct((n, K), jnp.int32),
        ],
        scratch_shapes=[pltpu.VMEM((2 * br, n), jnp.float32)],
    )(attr, maxval)

    values = vals.reshape(-1)
    rows = jnp.repeat(jnp.arange(n, dtype=jnp.int32), K)
    edge_index = jnp.stack([rows, idxs.reshape(-1)], axis=0)
    return values, edge_index
